# Initial kernel scaffold; baseline (speedup 1.0000x reference)
import jax
import jax.numpy as jnp
from jax.experimental import pallas as pl


def kernel(x, table):
    raise NotImplementedError("write your pallas kernel here")


# speedup vs baseline: 2.8003x; 2.8003x over previous
import functools
import jax
import jax.numpy as jnp
from jax import lax
from jax.experimental import pallas as pl
from jax.experimental.pallas import tpu as pltpu, tpu_sc as plsc

BATCH = 16384
HIST = 50
EMBED = 32

NC = 2
NS = 16
NW = NC * NS

B_PER_W = BATCH // NW
CB = 64
NCHUNK = B_PER_W // CB
ROWS_PER_CHUNK = CB * HIST
G = 128
NG = ROWS_PER_CHUNK // G
IDX_ROWS_PER_W = B_PER_W * HIST // G

_mesh = plsc.VectorSubcoreMesh(core_axis_name="c", subcore_axis_name="s")


@functools.partial(
    pl.kernel,
    out_type=jax.ShapeDtypeStruct((BATCH, EMBED), jnp.float32),
    mesh=_mesh,
    compiler_params=pltpu.CompilerParams(use_tc_tiling_on_sc=False),
    scratch_types=[
        pltpu.VMEM((ROWS_PER_CHUNK,), jnp.int32),
        pltpu.VMEM((ROWS_PER_CHUNK, EMBED), jnp.float32),
        pltpu.VMEM((CB, EMBED), jnp.float32),
        pltpu.SemaphoreType.DMA,
    ],
)
def _user_encoder(x_hbm, table_hbm, out_hbm, idx_v, rows_v, out_v, sem):
  wid = lax.axis_index("s") * NC + lax.axis_index("c")

  def chunk_body(c, _):
    i0 = (wid * B_PER_W + c * CB) * HIST
    pltpu.sync_copy(x_hbm.at[pl.ds(i0, ROWS_PER_CHUNK)], idx_v)

    def fire(j, _):
      pltpu.async_copy(table_hbm.at[idx_v.at[pl.ds(j * G, G)]],
                       rows_v.at[pl.ds(j * G, G)], sem)
      return 0
    lax.fori_loop(0, NG, fire, 0)

    def drain(j, _):
      pltpu.make_async_copy(table_hbm.at[idx_v.at[pl.ds(j * G, G)]],
                            rows_v.at[pl.ds(j * G, G)], sem).wait()
      return 0
    lax.fori_loop(0, NG, drain, 0)

    def pool(i, _):
      base = i * HIST
      acc0 = rows_v[base, 0:16]
      acc1 = rows_v[base, 16:32]
      for l in range(1, HIST):
        acc0 = acc0 + rows_v[base + l, 0:16]
        acc1 = acc1 + rows_v[base + l, 16:32]
      scale = jnp.float32(1.0 / HIST)
      out_v[i, 0:16] = acc0 * scale
      out_v[i, 16:32] = acc1 * scale
      return 0
    lax.fori_loop(0, CB, pool, 0)

    pltpu.sync_copy(out_v, out_hbm.at[pl.ds(wid * B_PER_W + c * CB, CB)])
    return 0

  lax.fori_loop(0, NCHUNK, chunk_body, 0)


def kernel(x, table):
  x_flat = x.astype(jnp.int32).reshape(BATCH * HIST)
  return _user_encoder(x_flat, table)


# speedup vs baseline: 2.8489x; 1.0174x over previous
import functools
import jax
import jax.numpy as jnp
from jax import lax
from jax.experimental import pallas as pl
from jax.experimental.pallas import tpu as pltpu, tpu_sc as plsc

BATCH = 16384
HIST = 50
EMBED = 32
DICT = 1000000

NC = 2
NS = 16
NW = NC * NS

B_PER_W = BATCH // NW
CB = 64
NCHUNK = B_PER_W // CB

_mesh = plsc.VectorSubcoreMesh(core_axis_name="c", subcore_axis_name="s")


@functools.partial(
    pl.kernel,
    out_type=jax.ShapeDtypeStruct((BATCH, EMBED), jnp.float32),
    mesh=_mesh,
    compiler_params=pltpu.CompilerParams(use_tc_tiling_on_sc=False),
    scratch_types=[
        pltpu.VMEM((HIST, B_PER_W), jnp.int32),
        pltpu.VMEM((HIST, CB, EMBED), jnp.float32),
        pltpu.VMEM((CB, EMBED), jnp.float32),
        pltpu.SemaphoreType.DMA,
    ],
)
def _user_encoder(xt_hbm, table_hbm, out_hbm, idx_v, rows_v, out_v, sem):
  wid = lax.axis_index("s") * NC + lax.axis_index("c")

  pltpu.sync_copy(xt_hbm.at[:, pl.ds(wid * B_PER_W, B_PER_W)], idx_v)

  def chunk_body(c, _):
    b0 = c * CB

    def fire(l, _):
      pltpu.async_copy(table_hbm.at[idx_v.at[l, pl.ds(b0, CB)]],
                       rows_v.at[l], sem)
      return 0
    lax.fori_loop(0, HIST, fire, 0)

    def drain(l, _):
      pltpu.make_async_copy(table_hbm.at[idx_v.at[l, pl.ds(b0, CB)]],
                            rows_v.at[l], sem).wait()
      return 0
    lax.fori_loop(0, HIST, drain, 0)

    def pool(i, _):
      acc0 = rows_v[0, i, 0:16]
      acc1 = rows_v[0, i, 16:32]
      for l in range(1, HIST):
        acc0 = acc0 + rows_v[l, i, 0:16]
        acc1 = acc1 + rows_v[l, i, 16:32]
      scale = jnp.float32(1.0 / HIST)
      out_v[i, 0:16] = acc0 * scale
      out_v[i, 16:32] = acc1 * scale
      return 0
    lax.fori_loop(0, CB, pool, 0)

    pltpu.sync_copy(out_v, out_hbm.at[pl.ds(wid * B_PER_W + b0, CB)])
    return 0

  lax.fori_loop(0, NCHUNK, chunk_body, 0)


def kernel(x, table):
  xt = x.astype(jnp.int32).T
  return _user_encoder(xt, table)
